# Initial kernel scaffold; baseline (speedup 1.0000x reference)
import jax
import jax.numpy as jnp
from jax.experimental import pallas as pl


def kernel(x_a, x_b, x_c, edge_ab, edge_bc, t_a, t_b, t_c, w_l0_W, w_l0_b, w_00_W, w_00_b, w_10_W, w_10_b, gate0, lam0, ln0_g, ln0_b, w_l1_W, w_l1_b, w_01_W, w_01_b, w_11_W, w_11_b, gate1, lam1, ln1_g, ln1_b, out_W, out_b):
    raise NotImplementedError("write your pallas kernel here")


# speedup vs baseline: 1.1772x; 1.1772x over previous
import functools

import jax
import jax.numpy as jnp
from jax import lax
from jax.experimental import pallas as pl
from jax.experimental.pallas import tpu as pltpu
from jax.experimental.pallas import tpu_sc as plsc

N = 10000
E = 320000
H = 128
N2 = 10240

NC, NS, L = 2, 16, 16
CH = 128
E2 = 327680
PAD_E = E2 - E
EPC = E2 // NC
EPT = EPC // NS
NCHUNK = EPT // CH
RPT = N2 // NS

_DIAG_EW = False


def _sc_body(e0_hbm, e1_hbm, xb_hbm, tb_hbm, tc_hbm, nlam_hbm,
                  zrow_hbm, zdeg_hbm, ew_hbm, agg_out, deg_out,
                  idx0_v, idx1_v, rows_v, ew_v, tb_v, tc_v, nlam_v,
                  agg_sh, deg_sh, sem):
    c = lax.axis_index("c")
    s = lax.axis_index("s")
    pltpu.sync_copy(tb_hbm, tb_v.at[pl.ds(0, N)])
    pltpu.sync_copy(tc_hbm, tc_v.at[pl.ds(0, N)])
    pltpu.sync_copy(zdeg_hbm.at[pl.ds(0, N2 - N)], tb_v.at[pl.ds(N, N2 - N)])
    pltpu.sync_copy(zdeg_hbm.at[pl.ds(0, N2 - N)], tc_v.at[pl.ds(N, N2 - N)])
    pltpu.sync_copy(nlam_hbm, nlam_v)
    pltpu.sync_copy(zrow_hbm, agg_sh.at[pl.ds(s * RPT, RPT)])
    pltpu.sync_copy(zdeg_hbm, deg_sh.at[pl.ds(s * RPT, RPT)])
    plsc.subcore_barrier()
    nlam = nlam_v[...]

    def chunk(gi, carry):
        base = c * EPC + s * EPT + gi * CH
        pltpu.sync_copy(e0_hbm.at[pl.ds(base, CH)], idx0_v)
        pltpu.sync_copy(e1_hbm.at[pl.ds(base, CH)], idx1_v)
        pltpu.async_copy(xb_hbm.at[idx0_v], rows_v, sem).wait()
        if _DIAG_EW:
            pltpu.sync_copy(ew_hbm.at[pl.ds(base, CH)], ew_v)
        else:
            for i in range(CH // L):
                i0 = idx0_v[pl.ds(i * L, L)]
                i1 = idx1_v[pl.ds(i * L, L)]
                t0 = plsc.load_gather(tb_v, [i0])
                t1 = plsc.load_gather(tc_v, [i1])
                dlt = jnp.maximum(t1 - t0, 0.0)
                z = jnp.maximum(nlam * dlt, -60.0)
                ew_v[pl.ds(i * L, L)] = jnp.exp(z)
        for e in range(CH):
            w = plsc.load_gather(ew_v, [jnp.full((L,), e, jnp.int32)])
            for k in range(H // L):
                rows_v[e, pl.ds(k * L, L)] = rows_v[e, pl.ds(k * L, L)] * w
        pltpu.sync_copy(ew_v, deg_sh.at[idx1_v], add=True)
        pltpu.sync_copy(rows_v, agg_sh.at[idx1_v], add=True)
        return carry

    lax.fori_loop(0, NCHUNK, chunk, 0)
    plsc.subcore_barrier()
    pltpu.sync_copy(agg_sh.at[pl.ds(s * RPT, RPT)],
                    agg_out.at[pl.ds(c * N2 + s * RPT, RPT)])
    pltpu.sync_copy(deg_sh.at[pl.ds(s * RPT, RPT)],
                    deg_out.at[pl.ds(c * N2 + s * RPT, RPT)])


@functools.cache
def _get_sc_aggregate():
    mesh = plsc.VectorSubcoreMesh(core_axis_name="c", subcore_axis_name="s",
                                  num_cores=NC, num_subcores=NS)
    return pl.kernel(
        _sc_body,
        out_type=[
            jax.ShapeDtypeStruct((NC * N2, H), jnp.float32),
            jax.ShapeDtypeStruct((NC * N2,), jnp.float32),
        ],
        mesh=mesh,
        compiler_params=pltpu.CompilerParams(needs_layout_passes=False),
        scratch_types=[
            pltpu.VMEM((CH,), jnp.int32),
            pltpu.VMEM((CH,), jnp.int32),
            pltpu.VMEM((CH, H), jnp.float32),
            pltpu.VMEM((CH,), jnp.float32),
            pltpu.VMEM((N2,), jnp.float32),
            pltpu.VMEM((N2,), jnp.float32),
            pltpu.VMEM((L,), jnp.float32),
            pltpu.VMEM_SHARED((N2, H), jnp.float32),
            pltpu.VMEM_SHARED((N2,), jnp.float32),
            pltpu.SemaphoreType.DMA,
        ],
    )


_R = 512


def _tc_body(agg_ref, d0_ref, d1_ref, xc_ref, wl_ref, wc_ref, wo_ref,
             bc_ref, lng_ref, lnb_ref, bo_ref, o_ref):
    a = agg_ref[0] + agg_ref[1]
    d = (d0_ref[...] + d1_ref[...])[:, None]
    pres = d > 0.0
    recip = 1.0 / jnp.maximum(d, 1e-6)
    xc = xc_ref[...]
    y = jnp.dot(a * recip, wl_ref[...], preferred_element_type=jnp.float32, precision=jax.lax.Precision.DEFAULT)
    y = y + jnp.dot(xc, wc_ref[...], preferred_element_type=jnp.float32, precision=jax.lax.Precision.DEFAULT)
    y = y + bc_ref[...][None, :]
    y = jnp.maximum(y, 0.0)
    mu = jnp.mean(y, axis=1, keepdims=True)
    var = jnp.mean((y - mu) ** 2, axis=1, keepdims=True)
    yn = (y - mu) / jnp.sqrt(var + 1e-5)
    yn = yn * lng_ref[...][None, :] + lnb_ref[...][None, :]
    h = jnp.where(pres, yn, xc)
    o_ref[...] = (jnp.dot(h, wo_ref[...], preferred_element_type=jnp.float32, precision=jax.lax.Precision.DEFAULT)
                  + bo_ref[...][None, :])


_tc_epilogue = pl.pallas_call(
    _tc_body,
    grid=(N2 // _R,),
    in_specs=[
        pl.BlockSpec((2, _R, H), lambda i: (0, i, 0)),
        pl.BlockSpec((_R,), lambda i: (i,)),
        pl.BlockSpec((_R,), lambda i: (i,)),
        pl.BlockSpec((_R, H), lambda i: (i, 0)),
        pl.BlockSpec((H, H), lambda i: (0, 0)),
        pl.BlockSpec((H, H), lambda i: (0, 0)),
        pl.BlockSpec((H, H), lambda i: (0, 0)),
        pl.BlockSpec((H,), lambda i: (0,)),
        pl.BlockSpec((H,), lambda i: (0,)),
        pl.BlockSpec((H,), lambda i: (0,)),
        pl.BlockSpec((H,), lambda i: (0,)),
    ],
    out_specs=pl.BlockSpec((_R, H), lambda i: (i, 0)),
    out_shape=jax.ShapeDtypeStruct((N2, H), jnp.float32),
)


def kernel(x_a, x_b, x_c, edge_ab, edge_bc, t_a, t_b, t_c,
           w_l0_W, w_l0_b, w_00_W, w_00_b, w_10_W, w_10_b,
           gate0, lam0, ln0_g, ln0_b,
           w_l1_W, w_l1_b, w_01_W, w_01_b, w_11_W, w_11_b,
           gate1, lam1, ln1_g, ln1_b,
           out_W, out_b):
    e0 = edge_bc[0].astype(jnp.int32)
    e1 = edge_bc[1].astype(jnp.int32)
    pad_i = jnp.arange(PAD_E, dtype=jnp.int32)
    e0 = jnp.concatenate([e0, pad_i % N])
    e1 = jnp.concatenate([e1, N + pad_i % (N2 - N)])
    lam = jax.nn.softplus(lam1) + 1e-8
    nlam = jnp.full((L,), -lam, dtype=jnp.float32)
    zrow = jnp.zeros((RPT, H), dtype=jnp.float32)
    zdeg = jnp.zeros((RPT,), dtype=jnp.float32)

    lam_ = jax.nn.softplus(lam1) + 1e-8
    delta = jnp.clip(t_c[e1] - t_b[e0], 0.0, None)
    ew_x = jnp.exp(jnp.clip(-lam_ * delta, -60.0, None))
    DIAG_SCATTER_ONLY = False
    if DIAG_SCATTER_ONLY:
        msgs = jnp.where(jnp.arange(E2)[:, None] < E, 1.0, 0.0) * (
            x_b[jnp.minimum(e0, N - 1)] * ew_x[:, None])
        agg_flat, deg_flat = _get_sc_aggregate()(
            jnp.arange(E2, dtype=jnp.int32), e1, msgs, t_b, t_c, nlam,
            zrow, zdeg, jnp.ones((E2,), jnp.float32))
    else:
        agg_flat, deg_flat = _get_sc_aggregate()(
            e0, e1, x_b, t_b, t_c, nlam, zrow, zdeg, ew_x)

    g = jax.nn.sigmoid(gate1)
    wl_t = w_l1_W.T
    wc_t = ((1.0 - g) * w_01_W + g * w_11_W).T
    wo_t = out_W.T
    b_comb = w_l1_b + (1.0 - g) * w_01_b + g * w_11_b

    agg = agg_flat.reshape(NC, N2, H)
    DIAG_DEG = False
    if DIAG_DEG:
        d0 = jnp.zeros((N2,)).at[e1[:E]].add(ew_x[:E])
        d1 = jnp.zeros((N2,))
    else:
        d0 = deg_flat[:N2]
        d1 = deg_flat[N2:]
    xc_pad = jnp.pad(x_c, ((0, N2 - N), (0, 0)))

    out = _tc_epilogue(agg, d0, d1, xc_pad, wl_t, wc_t, wo_t,
                       b_comb, ln1_g, ln1_b, out_b)
    return out[:N]


# speedup vs baseline: 13.9903x; 11.8842x over previous
import functools

import jax
import jax.numpy as jnp
from jax import lax
from jax.experimental import pallas as pl
from jax.experimental.pallas import tpu as pltpu
from jax.experimental.pallas import tpu_sc as plsc

N = 10000
E = 320000
H = 128
N2 = 10240

NC, NS, L = 2, 16, 16
CH = 128
E2 = 327680
PAD_E = E2 - E
EPC = E2 // NC
EPT = EPC // NS
NCHUNK = EPT // CH
RPT = N2 // NS


def _sc_body(e0_hbm, e1_hbm, xb_hbm, tb_hbm, tc_hbm, nlam_hbm,
             zrow_hbm, zdeg_hbm, agg_out, deg_out,
             idx0_v, idx1_v, rows_v, ew_v, tb_v, tc_v, nlam_v,
             agg_sh, deg_sh, sem):
    c = lax.axis_index("c")
    s = lax.axis_index("s")
    pltpu.sync_copy(tb_hbm, tb_v.at[pl.ds(0, N)])
    pltpu.sync_copy(tc_hbm, tc_v.at[pl.ds(0, N)])
    pltpu.sync_copy(zdeg_hbm.at[pl.ds(0, N2 - N)], tb_v.at[pl.ds(N, N2 - N)])
    pltpu.sync_copy(zdeg_hbm.at[pl.ds(0, N2 - N)], tc_v.at[pl.ds(N, N2 - N)])
    pltpu.sync_copy(nlam_hbm, nlam_v)
    pltpu.sync_copy(zrow_hbm, agg_sh.at[pl.ds(s * RPT, RPT)])
    pltpu.sync_copy(zdeg_hbm, deg_sh.at[pl.ds(s * RPT, RPT)])
    plsc.subcore_barrier()
    nlam = nlam_v[...]

    def chunk(gi, carry):
        base = c * EPC + s * EPT + gi * CH
        pltpu.sync_copy(e0_hbm.at[pl.ds(base, CH)], idx0_v)
        pltpu.sync_copy(e1_hbm.at[pl.ds(base, CH)], idx1_v)
        pltpu.async_copy(xb_hbm.at[idx0_v], rows_v, sem).wait()
        for i in range(CH // L):
            i0 = idx0_v[pl.ds(i * L, L)]
            i1 = idx1_v[pl.ds(i * L, L)]
            t0 = plsc.load_gather(tb_v, [i0])
            t1 = plsc.load_gather(tc_v, [i1])
            dlt = jnp.maximum(t1 - t0, 0.0)
            z = jnp.maximum(nlam * dlt, -60.0)
            ew_v[pl.ds(i * L, L)] = jnp.exp(z)
        for e in range(CH):
            w = plsc.load_gather(ew_v, [jnp.full((L,), e, jnp.int32)])
            for k in range(H // L):
                rows_v[e, pl.ds(k * L, L)] = rows_v[e, pl.ds(k * L, L)] * w
        pltpu.sync_copy(ew_v, deg_sh.at[idx1_v], add=True)
        pltpu.sync_copy(rows_v, agg_sh.at[idx1_v], add=True)
        return carry

    lax.fori_loop(0, NCHUNK, chunk, 0)
    plsc.subcore_barrier()
    pltpu.sync_copy(agg_sh.at[pl.ds(s * RPT, RPT)],
                    agg_out.at[pl.ds(c * N2 + s * RPT, RPT)])
    pltpu.sync_copy(deg_sh.at[pl.ds(s * RPT, RPT)],
                    deg_out.at[pl.ds(c * N2 + s * RPT, RPT)])


@functools.cache
def _get_sc_aggregate():
    mesh = plsc.VectorSubcoreMesh(core_axis_name="c", subcore_axis_name="s",
                                  num_cores=NC, num_subcores=NS)
    return pl.kernel(
        _sc_body,
        out_type=[
            jax.ShapeDtypeStruct((NC * N2, H), jnp.float32),
            jax.ShapeDtypeStruct((NC * N2,), jnp.float32),
        ],
        mesh=mesh,
        compiler_params=pltpu.CompilerParams(needs_layout_passes=False),
        scratch_types=[
            pltpu.VMEM((CH,), jnp.int32),
            pltpu.VMEM((CH,), jnp.int32),
            pltpu.VMEM((CH, H), jnp.float32),
            pltpu.VMEM((CH,), jnp.float32),
            pltpu.VMEM((N2,), jnp.float32),
            pltpu.VMEM((N2,), jnp.float32),
            pltpu.VMEM((L,), jnp.float32),
            pltpu.VMEM_SHARED((N2, H), jnp.float32),
            pltpu.VMEM_SHARED((N2,), jnp.float32),
            pltpu.SemaphoreType.DMA,
        ],
    )


_R = 512


def _tc_body(agg_ref, d0_ref, d1_ref, xc_ref, wl_ref, wc_ref, wo_ref,
             bc_ref, lng_ref, lnb_ref, bo_ref, o_ref):
    a = agg_ref[0] + agg_ref[1]
    d = (d0_ref[...] + d1_ref[...])[:, None]
    pres = d > 0.0
    recip = 1.0 / jnp.maximum(d, 1e-6)
    xc = xc_ref[...]
    y = jnp.dot(a * recip, wl_ref[...], preferred_element_type=jnp.float32)
    y = y + jnp.dot(xc, wc_ref[...], preferred_element_type=jnp.float32)
    y = y + bc_ref[...][None, :]
    y = jnp.maximum(y, 0.0)
    mu = jnp.mean(y, axis=1, keepdims=True)
    var = jnp.mean((y - mu) ** 2, axis=1, keepdims=True)
    yn = (y - mu) / jnp.sqrt(var + 1e-5)
    yn = yn * lng_ref[...][None, :] + lnb_ref[...][None, :]
    h = jnp.where(pres, yn, xc)
    o_ref[...] = (jnp.dot(h, wo_ref[...], preferred_element_type=jnp.float32)
                  + bo_ref[...][None, :])


_tc_epilogue = pl.pallas_call(
    _tc_body,
    grid=(N2 // _R,),
    in_specs=[
        pl.BlockSpec((2, _R, H), lambda i: (0, i, 0)),
        pl.BlockSpec((_R,), lambda i: (i,)),
        pl.BlockSpec((_R,), lambda i: (i,)),
        pl.BlockSpec((_R, H), lambda i: (i, 0)),
        pl.BlockSpec((H, H), lambda i: (0, 0)),
        pl.BlockSpec((H, H), lambda i: (0, 0)),
        pl.BlockSpec((H, H), lambda i: (0, 0)),
        pl.BlockSpec((H,), lambda i: (0,)),
        pl.BlockSpec((H,), lambda i: (0,)),
        pl.BlockSpec((H,), lambda i: (0,)),
        pl.BlockSpec((H,), lambda i: (0,)),
    ],
    out_specs=pl.BlockSpec((_R, H), lambda i: (i, 0)),
    out_shape=jax.ShapeDtypeStruct((N2, H), jnp.float32),
)


def kernel(x_a, x_b, x_c, edge_ab, edge_bc, t_a, t_b, t_c,
           w_l0_W, w_l0_b, w_00_W, w_00_b, w_10_W, w_10_b,
           gate0, lam0, ln0_g, ln0_b,
           w_l1_W, w_l1_b, w_01_W, w_01_b, w_11_W, w_11_b,
           gate1, lam1, ln1_g, ln1_b,
           out_W, out_b):
    e0 = edge_bc[0].astype(jnp.int32)
    e1 = edge_bc[1].astype(jnp.int32)
    pad_i = jnp.arange(PAD_E, dtype=jnp.int32)
    e0 = jnp.concatenate([e0, pad_i % N])
    e1 = jnp.concatenate([e1, N + pad_i % (N2 - N)])
    lam = jax.nn.softplus(lam1) + 1e-8
    nlam = jnp.full((L,), -lam, dtype=jnp.float32)
    zrow = jnp.zeros((RPT, H), dtype=jnp.float32)
    zdeg = jnp.zeros((RPT,), dtype=jnp.float32)

    agg_flat, deg_flat = _get_sc_aggregate()(
        e0, e1, x_b, t_b, t_c, nlam, zrow, zdeg)

    g = jax.nn.sigmoid(gate1)
    wl_t = w_l1_W.T
    wc_t = ((1.0 - g) * w_01_W + g * w_11_W).T
    wo_t = out_W.T
    b_comb = w_l1_b + (1.0 - g) * w_01_b + g * w_11_b

    agg = agg_flat.reshape(NC, N2, H)
    d0 = deg_flat[:N2]
    d1 = deg_flat[N2:]
    xc_pad = jnp.pad(x_c, ((0, N2 - N), (0, 0)))

    out = _tc_epilogue(agg, d0, d1, xc_pad, wl_t, wc_t, wo_t,
                       b_comb, ln1_g, ln1_b, out_b)
    return out[:N]


# speedup vs baseline: 15.5065x; 1.1084x over previous
import functools

import jax
import jax.numpy as jnp
from jax import lax
from jax.experimental import pallas as pl
from jax.experimental.pallas import tpu as pltpu
from jax.experimental.pallas import tpu_sc as plsc

N = 10000
E = 320000
H = 128
N2 = 10240

NC, NS, L = 2, 16, 16
CH = 128
E2 = 327680
PAD_E = E2 - E
EPC = E2 // NC
EPT = EPC // NS
NCHUNK = EPT // CH
RPT = N2 // NS


def _sc_body(e0_hbm, e1_hbm, xb_hbm, tb_hbm, tc_hbm, nlam_hbm,
             agg_out, deg_out,
             idx0_v, idx1_v, sidx_v, rows_v, wrows_v, ew_v, t0_v, t1_v,
             nlam_v, agg_sh, deg_sh, gsem, ssem, dsem, tsem):
    c = lax.axis_index("c")
    s = lax.axis_index("s")
    pltpu.sync_copy(nlam_hbm, nlam_v)
    z16 = jnp.zeros((L,), jnp.float32)
    for e in range(CH):
        for k in range(H // L):
            wrows_v[e, pl.ds(k * L, L)] = z16
    for k in range(CH // L):
        ew_v[pl.ds(k * L, L)] = z16
    for j in range(RPT // CH):
        pltpu.sync_copy(wrows_v, agg_sh.at[pl.ds(s * RPT + j * CH, CH)])
        pltpu.sync_copy(ew_v, deg_sh.at[pl.ds(s * RPT + j * CH, CH)])
    plsc.subcore_barrier()
    nlam = nlam_v[...]

    def chunk(gi, carry):
        base = c * EPC + s * EPT + gi * CH
        pltpu.sync_copy(e0_hbm.at[pl.ds(base, CH)], idx0_v)
        pltpu.sync_copy(e1_hbm.at[pl.ds(base, CH)], idx1_v)
        dg = pltpu.async_copy(xb_hbm.at[idx0_v], rows_v, gsem)
        dt0 = pltpu.async_copy(tb_hbm.at[idx0_v], t0_v, tsem)
        dt1 = pltpu.async_copy(tc_hbm.at[idx1_v], t1_v, tsem)

        @pl.when(gi > 0)
        def _():
            pltpu.make_async_copy(wrows_v, agg_sh.at[sidx_v], ssem).wait()
            pltpu.make_async_copy(ew_v, deg_sh.at[sidx_v], dsem).wait()

        dt0.wait()
        dt1.wait()
        for i in range(CH // L):
            sidx_v[pl.ds(i * L, L)] = idx1_v[pl.ds(i * L, L)]
            t0 = t0_v[pl.ds(i * L, L)]
            t1 = t1_v[pl.ds(i * L, L)]
            dlt = jnp.maximum(t1 - t0, 0.0)
            z = jnp.maximum(nlam * dlt, -60.0)
            ew_v[pl.ds(i * L, L)] = jnp.exp(z)
        dg.wait()
        for e in range(CH):
            w = plsc.load_gather(ew_v, [jnp.full((L,), e, jnp.int32)])
            for k in range(H // L):
                wrows_v[e, pl.ds(k * L, L)] = rows_v[e, pl.ds(k * L, L)] * w
        pltpu.async_copy(wrows_v, agg_sh.at[sidx_v], ssem, add=True)
        pltpu.async_copy(ew_v, deg_sh.at[sidx_v], dsem, add=True)
        return carry

    lax.fori_loop(0, NCHUNK, chunk, 0)
    pltpu.make_async_copy(wrows_v, agg_sh.at[sidx_v], ssem).wait()
    pltpu.make_async_copy(ew_v, deg_sh.at[sidx_v], dsem).wait()
    plsc.subcore_barrier()
    pltpu.sync_copy(agg_sh.at[pl.ds(s * RPT, RPT)],
                    agg_out.at[pl.ds(c * N2 + s * RPT, RPT)])
    pltpu.sync_copy(deg_sh.at[pl.ds(s * RPT, RPT)],
                    deg_out.at[pl.ds(c * N2 + s * RPT, RPT)])


@functools.cache
def _get_sc_aggregate():
    mesh = plsc.VectorSubcoreMesh(core_axis_name="c", subcore_axis_name="s",
                                  num_cores=NC, num_subcores=NS)
    return pl.kernel(
        _sc_body,
        out_type=[
            jax.ShapeDtypeStruct((NC * N2, H), jnp.float32),
            jax.ShapeDtypeStruct((NC * N2,), jnp.float32),
        ],
        mesh=mesh,
        compiler_params=pltpu.CompilerParams(needs_layout_passes=False),
        scratch_types=[
            pltpu.VMEM((CH,), jnp.int32),
            pltpu.VMEM((CH,), jnp.int32),
            pltpu.VMEM((CH,), jnp.int32),
            pltpu.VMEM((CH, H), jnp.float32),
            pltpu.VMEM((CH, H), jnp.float32),
            pltpu.VMEM((CH,), jnp.float32),
            pltpu.VMEM((CH,), jnp.float32),
            pltpu.VMEM((CH,), jnp.float32),
            pltpu.VMEM((L,), jnp.float32),
            pltpu.VMEM_SHARED((N2, H), jnp.float32),
            pltpu.VMEM_SHARED((N2,), jnp.float32),
            pltpu.SemaphoreType.DMA,
            pltpu.SemaphoreType.DMA,
            pltpu.SemaphoreType.DMA,
            pltpu.SemaphoreType.DMA,
        ],
    )


_R = 512


def _tc_body(agg_ref, d0_ref, d1_ref, xc_ref, wl_ref, wc_ref, wo_ref,
             bc_ref, lng_ref, lnb_ref, bo_ref, o_ref):
    a = agg_ref[0] + agg_ref[1]
    d = (d0_ref[...] + d1_ref[...])[:, None]
    pres = d > 0.0
    recip = 1.0 / jnp.maximum(d, 1e-6)
    xc = xc_ref[...]
    y = jnp.dot(a * recip, wl_ref[...], preferred_element_type=jnp.float32)
    y = y + jnp.dot(xc, wc_ref[...], preferred_element_type=jnp.float32)
    y = y + bc_ref[...][None, :]
    y = jnp.maximum(y, 0.0)
    mu = jnp.mean(y, axis=1, keepdims=True)
    var = jnp.mean((y - mu) ** 2, axis=1, keepdims=True)
    yn = (y - mu) / jnp.sqrt(var + 1e-5)
    yn = yn * lng_ref[...][None, :] + lnb_ref[...][None, :]
    h = jnp.where(pres, yn, xc)
    o_ref[...] = (jnp.dot(h, wo_ref[...], preferred_element_type=jnp.float32)
                  + bo_ref[...][None, :])


_tc_epilogue = pl.pallas_call(
    _tc_body,
    grid=(N2 // _R,),
    in_specs=[
        pl.BlockSpec((2, _R, H), lambda i: (0, i, 0)),
        pl.BlockSpec((_R,), lambda i: (i,)),
        pl.BlockSpec((_R,), lambda i: (i,)),
        pl.BlockSpec((_R, H), lambda i: (i, 0)),
        pl.BlockSpec((H, H), lambda i: (0, 0)),
        pl.BlockSpec((H, H), lambda i: (0, 0)),
        pl.BlockSpec((H, H), lambda i: (0, 0)),
        pl.BlockSpec((H,), lambda i: (0,)),
        pl.BlockSpec((H,), lambda i: (0,)),
        pl.BlockSpec((H,), lambda i: (0,)),
        pl.BlockSpec((H,), lambda i: (0,)),
    ],
    out_specs=pl.BlockSpec((_R, H), lambda i: (i, 0)),
    out_shape=jax.ShapeDtypeStruct((N2, H), jnp.float32),
)


def kernel(x_a, x_b, x_c, edge_ab, edge_bc, t_a, t_b, t_c,
           w_l0_W, w_l0_b, w_00_W, w_00_b, w_10_W, w_10_b,
           gate0, lam0, ln0_g, ln0_b,
           w_l1_W, w_l1_b, w_01_W, w_01_b, w_11_W, w_11_b,
           gate1, lam1, ln1_g, ln1_b,
           out_W, out_b):
    e0 = edge_bc[0].astype(jnp.int32)
    e1 = edge_bc[1].astype(jnp.int32)
    pad_i = jnp.arange(PAD_E, dtype=jnp.int32)
    e0 = jnp.concatenate([e0, pad_i % N])
    e1 = jnp.concatenate([e1, N + pad_i % (N2 - N)])
    lam = jax.nn.softplus(lam1) + 1e-8
    nlam = jnp.full((L,), -lam, dtype=jnp.float32)
    tbp = jnp.pad(t_b, (0, N2 - N))
    tcp = jnp.pad(t_c, (0, N2 - N))
    agg_flat, deg_flat = _get_sc_aggregate()(e0, e1, x_b, tbp, tcp, nlam)

    g = jax.nn.sigmoid(gate1)
    wl_t = w_l1_W.T
    wc_t = ((1.0 - g) * w_01_W + g * w_11_W).T
    wo_t = out_W.T
    b_comb = w_l1_b + (1.0 - g) * w_01_b + g * w_11_b

    agg = agg_flat.reshape(NC, N2, H)
    d0 = deg_flat[:N2]
    d1 = deg_flat[N2:]
    xc_pad = jnp.pad(x_c, ((0, N2 - N), (0, 0)))

    out = _tc_epilogue(agg, d0, d1, xc_pad, wl_t, wc_t, wo_t,
                       b_comb, ln1_g, ln1_b, out_b)
    return out[:N]


# speedup vs baseline: 17.5275x; 1.1303x over previous
import functools

import jax
import jax.numpy as jnp
from jax import lax
from jax.experimental import pallas as pl
from jax.experimental.pallas import tpu as pltpu
from jax.experimental.pallas import tpu_sc as plsc

N = 10000
E = 320000
H = 128
N2 = 10240

NC, NS, L = 2, 16, 16
CH = 128
E2 = 327680
PAD_E = E2 - E
EPC = E2 // NC
EPT = EPC // NS
NCHUNK = EPT // CH
RPT = N2 // NS


def _sc_body(e0_hbm, e1_hbm, xb_hbm, tb_hbm, tc_hbm, nlam_hbm,
             agg_out, deg_out,
             idx0_v, idx1_v, sidx_v, rows_v, wrows_v, ew_v, t0_v, t1_v,
             nlam_v, agg_sh, deg_sh, gsem, ssem, dsem, tsem, isem):
    c = lax.axis_index("c")
    s = lax.axis_index("s")
    pltpu.sync_copy(nlam_hbm, nlam_v)
    z16 = jnp.zeros((L,), jnp.float32)
    for e in range(CH):
        for k in range(H // L):
            wrows_v[e, pl.ds(k * L, L)] = z16
    for k in range(CH // L):
        ew_v[pl.ds(k * L, L)] = z16
    for j in range(RPT // CH):
        pltpu.sync_copy(wrows_v, agg_sh.at[pl.ds(s * RPT + j * CH, CH)])
        pltpu.sync_copy(ew_v, deg_sh.at[pl.ds(s * RPT + j * CH, CH)])
    plsc.subcore_barrier()
    nlam = nlam_v[...]
    ebase0 = c * EPC + s * EPT
    pltpu.sync_copy(e0_hbm.at[pl.ds(ebase0, CH)], idx0_v)
    pltpu.sync_copy(e1_hbm.at[pl.ds(ebase0, CH)], idx1_v)

    def chunk(gi, carry):
        base = c * EPC + s * EPT + gi * CH

        @pl.when(gi > 0)
        def _():
            pltpu.make_async_copy(e0_hbm.at[pl.ds(base, CH)], idx0_v,
                                  isem).wait()
            pltpu.make_async_copy(e1_hbm.at[pl.ds(base, CH)], idx1_v,
                                  isem).wait()

        dg = pltpu.async_copy(xb_hbm.at[idx0_v], rows_v, gsem)
        dt0 = pltpu.async_copy(tb_hbm.at[idx0_v], t0_v, tsem)
        dt1 = pltpu.async_copy(tc_hbm.at[idx1_v], t1_v, tsem)

        @pl.when(gi > 0)
        def _():
            pltpu.make_async_copy(wrows_v, agg_sh.at[sidx_v], ssem).wait()
            pltpu.make_async_copy(ew_v, deg_sh.at[sidx_v], dsem).wait()

        dt0.wait()
        dt1.wait()
        for i in range(CH // L):
            sidx_v[pl.ds(i * L, L)] = idx1_v[pl.ds(i * L, L)]
            t0 = t0_v[pl.ds(i * L, L)]
            t1 = t1_v[pl.ds(i * L, L)]
            dlt = jnp.maximum(t1 - t0, 0.0)
            z = jnp.maximum(nlam * dlt, -60.0)
            ew_v[pl.ds(i * L, L)] = jnp.exp(z)
        dg.wait()

        @pl.when(gi < NCHUNK - 1)
        def _():
            nbase = base + CH
            pltpu.async_copy(e0_hbm.at[pl.ds(nbase, CH)], idx0_v, isem)
            pltpu.async_copy(e1_hbm.at[pl.ds(nbase, CH)], idx1_v, isem)

        for e in range(CH):
            w = plsc.load_gather(ew_v, [jnp.full((L,), e, jnp.int32)])
            for k in range(H // L):
                wrows_v[e, pl.ds(k * L, L)] = rows_v[e, pl.ds(k * L, L)] * w
        pltpu.async_copy(wrows_v, agg_sh.at[sidx_v], ssem, add=True)
        pltpu.async_copy(ew_v, deg_sh.at[sidx_v], dsem, add=True)
        return carry

    lax.fori_loop(0, NCHUNK, chunk, 0)
    pltpu.make_async_copy(wrows_v, agg_sh.at[sidx_v], ssem).wait()
    pltpu.make_async_copy(ew_v, deg_sh.at[sidx_v], dsem).wait()
    plsc.subcore_barrier()
    pltpu.sync_copy(agg_sh.at[pl.ds(s * RPT, RPT)],
                    agg_out.at[pl.ds(c * N2 + s * RPT, RPT)])
    pltpu.sync_copy(deg_sh.at[pl.ds(s * RPT, RPT)],
                    deg_out.at[pl.ds(c * N2 + s * RPT, RPT)])


@functools.cache
def _get_sc_aggregate():
    mesh = plsc.VectorSubcoreMesh(core_axis_name="c", subcore_axis_name="s",
                                  num_cores=NC, num_subcores=NS)
    return pl.kernel(
        _sc_body,
        out_type=[
            jax.ShapeDtypeStruct((NC * N2, H), jnp.float32),
            jax.ShapeDtypeStruct((NC * N2,), jnp.float32),
        ],
        mesh=mesh,
        compiler_params=pltpu.CompilerParams(needs_layout_passes=False),
        scratch_types=[
            pltpu.VMEM((CH,), jnp.int32),
            pltpu.VMEM((CH,), jnp.int32),
            pltpu.VMEM((CH,), jnp.int32),
            pltpu.VMEM((CH, H), jnp.float32),
            pltpu.VMEM((CH, H), jnp.float32),
            pltpu.VMEM((CH,), jnp.float32),
            pltpu.VMEM((CH,), jnp.float32),
            pltpu.VMEM((CH,), jnp.float32),
            pltpu.VMEM((L,), jnp.float32),
            pltpu.VMEM_SHARED((N2, H), jnp.float32),
            pltpu.VMEM_SHARED((N2,), jnp.float32),
            pltpu.SemaphoreType.DMA,
            pltpu.SemaphoreType.DMA,
            pltpu.SemaphoreType.DMA,
            pltpu.SemaphoreType.DMA,
            pltpu.SemaphoreType.DMA,
        ],
    )


_R = 512


def _tc_body(agg_ref, d0_ref, d1_ref, xc_ref, wl_ref, wc_ref, wo_ref,
             bc_ref, lng_ref, lnb_ref, bo_ref, o_ref):
    a = agg_ref[0] + agg_ref[1]
    d = (d0_ref[...] + d1_ref[...])[:, None]
    pres = d > 0.0
    recip = 1.0 / jnp.maximum(d, 1e-6)
    xc = xc_ref[...]
    y = jnp.dot(a * recip, wl_ref[...], preferred_element_type=jnp.float32)
    y = y + jnp.dot(xc, wc_ref[...], preferred_element_type=jnp.float32)
    y = y + bc_ref[...][None, :]
    y = jnp.maximum(y, 0.0)
    mu = jnp.mean(y, axis=1, keepdims=True)
    var = jnp.mean((y - mu) ** 2, axis=1, keepdims=True)
    yn = (y - mu) / jnp.sqrt(var + 1e-5)
    yn = yn * lng_ref[...][None, :] + lnb_ref[...][None, :]
    h = jnp.where(pres, yn, xc)
    o_ref[...] = (jnp.dot(h, wo_ref[...], preferred_element_type=jnp.float32)
                  + bo_ref[...][None, :])


_tc_epilogue = pl.pallas_call(
    _tc_body,
    grid=(N2 // _R,),
    in_specs=[
        pl.BlockSpec((2, _R, H), lambda i: (0, i, 0)),
        pl.BlockSpec((_R,), lambda i: (i,)),
        pl.BlockSpec((_R,), lambda i: (i,)),
        pl.BlockSpec((_R, H), lambda i: (i, 0)),
        pl.BlockSpec((H, H), lambda i: (0, 0)),
        pl.BlockSpec((H, H), lambda i: (0, 0)),
        pl.BlockSpec((H, H), lambda i: (0, 0)),
        pl.BlockSpec((H,), lambda i: (0,)),
        pl.BlockSpec((H,), lambda i: (0,)),
        pl.BlockSpec((H,), lambda i: (0,)),
        pl.BlockSpec((H,), lambda i: (0,)),
    ],
    out_specs=pl.BlockSpec((_R, H), lambda i: (i, 0)),
    out_shape=jax.ShapeDtypeStruct((N2, H), jnp.float32),
)


def kernel(x_a, x_b, x_c, edge_ab, edge_bc, t_a, t_b, t_c,
           w_l0_W, w_l0_b, w_00_W, w_00_b, w_10_W, w_10_b,
           gate0, lam0, ln0_g, ln0_b,
           w_l1_W, w_l1_b, w_01_W, w_01_b, w_11_W, w_11_b,
           gate1, lam1, ln1_g, ln1_b,
           out_W, out_b):
    e0 = edge_bc[0].astype(jnp.int32)
    e1 = edge_bc[1].astype(jnp.int32)
    pad_i = jnp.arange(PAD_E, dtype=jnp.int32)
    e0 = jnp.concatenate([e0, pad_i % N])
    e1 = jnp.concatenate([e1, N + pad_i % (N2 - N)])
    lam = jax.nn.softplus(lam1) + 1e-8
    nlam = jnp.full((L,), -lam, dtype=jnp.float32)
    tbp = jnp.pad(t_b, (0, N2 - N))
    tcp = jnp.pad(t_c, (0, N2 - N))
    agg_flat, deg_flat = _get_sc_aggregate()(e0, e1, x_b, tbp, tcp, nlam)

    g = jax.nn.sigmoid(gate1)
    wl_t = w_l1_W.T
    wc_t = ((1.0 - g) * w_01_W + g * w_11_W).T
    wo_t = out_W.T
    b_comb = w_l1_b + (1.0 - g) * w_01_b + g * w_11_b

    agg = agg_flat.reshape(NC, N2, H)
    d0 = deg_flat[:N2]
    d1 = deg_flat[N2:]
    xc_pad = jnp.pad(x_c, ((0, N2 - N), (0, 0)))

    out = _tc_epilogue(agg, d0, d1, xc_pad, wl_t, wc_t, wo_t,
                       b_comb, ln1_g, ln1_b, out_b)
    return out[:N]
